# baseline (speedup 1.0000x reference)
import functools

import jax
import jax.numpy as jnp
import numpy as np
from jax import lax
from jax.experimental import pallas as pl
from jax.experimental.pallas import tpu as pltpu
from jax.experimental.pallas import tpu_sc as plsc

VOCAB = 100000
D = 768
B = 4
S = 2048
N_ROWS = B * S

_info = plsc.get_sparse_core_info()
NC, NS, L = _info.num_cores, _info.num_subcores, _info.num_lanes
NW = NC * NS
POS_PER_W = S // NW
BLK = 8
NBLK = POS_PER_W // BLK
NGEN = 3
GROUPS = D // L

SCALE = np.float32(np.sqrt(np.float32(D)))

_mesh = plsc.VectorSubcoreMesh(core_axis_name="c", subcore_axis_name="s")


@functools.partial(
    pl.kernel,
    mesh=_mesh,
    out_type=jax.ShapeDtypeStruct((N_ROWS, D), jnp.float32),
    scratch_types=(
        [pltpu.VMEM((B * POS_PER_W,), jnp.int32)]
        + [pltpu.VMEM((BLK, D), jnp.float32) for _ in range(NGEN * (B + 1))]
        + [pltpu.SemaphoreType.DMA for _ in range(1 + 2 * NGEN)]
    ),
)
def _embed_kernel(
    idx_hbm, table_hbm, pos_hbm, out_hbm,
    idx_v,
    xa0, xa1, xa2, xa3, pa,
    xb0, xb1, xb2, xb3, pb,
    xc0, xc1, xc2, xc3, pc,
    isem, gin0, gin1, gin2, gout0, gout1, gout2,
):
    wid = lax.axis_index("s") * NC + lax.axis_index("c")
    pos0 = wid * POS_PER_W

    icps = [
        pltpu.async_copy(
            idx_hbm.at[b, pl.ds(pos0, POS_PER_W)],
            idx_v.at[pl.ds(b * POS_PER_W, POS_PER_W)],
            isem,
        )
        for b in range(B)
    ]

    xv = ((xa0, xa1, xa2, xa3), (xb0, xb1, xb2, xb3), (xc0, xc1, xc2, xc3))
    pv = (pa, pb, pc)
    gin = (gin0, gin1, gin2)
    gout = (gout0, gout1, gout2)

    def start_ins(g):
        p = g % NGEN
        cps = [
            pltpu.async_copy(
                pos_hbm.at[pl.ds(pos0 + g * BLK, BLK), :], pv[p], gin[p]
            )
        ]
        cps += [
            pltpu.async_copy(
                table_hbm.at[idx_v.at[pl.ds(b * POS_PER_W + g * BLK, BLK)]],
                xv[p][b],
                gin[p],
            )
            for b in range(B)
        ]
        return cps

    def start_outs(g):
        p = g % NGEN
        return [
            pltpu.async_copy(
                xv[p][b],
                out_hbm.at[pl.ds(b * S + pos0 + g * BLK, BLK), :],
                gout[p],
            )
            for b in range(B)
        ]

    for icp in icps:
        icp.wait()

    pending_in = {0: start_ins(0), 1: start_ins(1)}
    pending_out = {}

    for g in range(NBLK):
        p = g % NGEN
        for cp in pending_in.pop(g):
            cp.wait()
        if g + 1 < NBLK and g + 1 not in pending_in:
            q = (g + 1) % NGEN
            if q in pending_out:
                for cp in pending_out.pop(q):
                    cp.wait()
            pending_in[g + 1] = start_ins(g + 1)

        x0, x1, x2, x3 = xv[p]
        pos_v = pv[p]

        @plsc.parallel_loop(0, BLK, unroll=1)
        def row_body(r):
            @plsc.parallel_loop(0, D, step=L, unroll=4)
            def group_body(off):
                sl = pl.ds(off, L)
                pg = pos_v[r, sl]
                x0[r, sl] = x0[r, sl] * SCALE + pg
                x1[r, sl] = x1[r, sl] * SCALE + pg
                x2[r, sl] = x2[r, sl] * SCALE + pg
                x3[r, sl] = x3[r, sl] * SCALE + pg

        pending_out[p] = start_outs(g)

    for p in list(pending_out):
        for cp in pending_out.pop(p):
            cp.wait()


def kernel(inputs, token_table, pos_embedding):
    idx = inputs.astype(jnp.int32)
    out = _embed_kernel(idx, token_table, pos_embedding)
    return out.reshape(B, S, D)


# speedup vs baseline: 1.0051x; 1.0051x over previous
import functools

import jax
import jax.numpy as jnp
import numpy as np
from jax import lax
from jax.experimental import pallas as pl
from jax.experimental.pallas import tpu as pltpu
from jax.experimental.pallas import tpu_sc as plsc

VOCAB = 100000
D = 768
B = 4
S = 2048
N_ROWS = B * S

_info = plsc.get_sparse_core_info()
NC, NS, L = _info.num_cores, _info.num_subcores, _info.num_lanes
NW = NC * NS
POS_PER_W = S // NW
BLK = 8
NBLK = POS_PER_W // BLK
NGEN = 3
ROWS = B * BLK
GROUPS = D // L

SCALE = np.float32(np.sqrt(np.float32(D)))

_mesh = plsc.VectorSubcoreMesh(core_axis_name="c", subcore_axis_name="s")


@functools.partial(
    pl.kernel,
    mesh=_mesh,
    out_type=jax.ShapeDtypeStruct((N_ROWS, D), jnp.float32),
    scratch_types=(
        [pltpu.VMEM((NBLK * ROWS,), jnp.int32)]
        + [pltpu.VMEM((ROWS, D), jnp.float32) for _ in range(NGEN)]
        + [pltpu.VMEM((BLK, D), jnp.float32) for _ in range(NGEN)]
        + [pltpu.SemaphoreType.DMA for _ in range(1 + 2 * NGEN)]
    ),
)
def _embed_kernel(
    idx_hbm, table_hbm, pos_hbm, out_hbm,
    idx_v, xa, xb, xc, pa, pb, pc,
    isem, gin0, gin1, gin2, gout0, gout1, gout2,
):
    wid = lax.axis_index("s") * NC + lax.axis_index("c")
    pos0 = wid * POS_PER_W

    icp = pltpu.async_copy(
        idx_hbm.at[pl.ds(wid * (NBLK * ROWS), NBLK * ROWS)], idx_v, isem
    )

    xv = (xa, xb, xc)
    pv = (pa, pb, pc)
    gin = (gin0, gin1, gin2)
    gout = (gout0, gout1, gout2)

    def start_ins(g):
        p = g % NGEN
        return [
            pltpu.async_copy(
                pos_hbm.at[pl.ds(pos0 + g * BLK, BLK), :], pv[p], gin[p]
            ),
            pltpu.async_copy(
                table_hbm.at[idx_v.at[pl.ds(g * ROWS, ROWS)]], xv[p], gin[p]
            ),
        ]

    def start_outs(g):
        p = g % NGEN
        return [
            pltpu.async_copy(
                xv[p].at[pl.ds(b * BLK, BLK), :],
                out_hbm.at[pl.ds(b * S + pos0 + g * BLK, BLK), :],
                gout[p],
            )
            for b in range(B)
        ]

    icp.wait()

    pending_in = {0: start_ins(0), 1: start_ins(1)}
    pending_out = {}

    for g in range(NBLK):
        p = g % NGEN
        for cp in pending_in.pop(g):
            cp.wait()
        if g + 1 < NBLK and g + 1 not in pending_in:
            q = (g + 1) % NGEN
            if q in pending_out:
                for cp in pending_out.pop(q):
                    cp.wait()
            pending_in[g + 1] = start_ins(g + 1)

        x_v = xv[p]
        pos_v = pv[p]

        @plsc.parallel_loop(0, BLK, unroll=1)
        def row_body(r):
            @plsc.parallel_loop(0, D, step=L, unroll=4)
            def group_body(off):
                sl = pl.ds(off, L)
                pg = pos_v[r, sl]
                x_v[r, sl] = x_v[r, sl] * SCALE + pg
                x_v[BLK + r, sl] = x_v[BLK + r, sl] * SCALE + pg
                x_v[2 * BLK + r, sl] = x_v[2 * BLK + r, sl] * SCALE + pg
                x_v[3 * BLK + r, sl] = x_v[3 * BLK + r, sl] * SCALE + pg

        pending_out[p] = start_outs(g)

    for p in list(pending_out):
        for cp in pending_out.pop(p):
            cp.wait()


def kernel(inputs, token_table, pos_embedding):
    idx = (
        inputs.astype(jnp.int32)
        .reshape(B, NW, NBLK, BLK)
        .transpose(1, 2, 0, 3)
        .reshape(N_ROWS)
    )
    out = _embed_kernel(idx, token_table, pos_embedding)
    return out.reshape(B, S, D)


# speedup vs baseline: 1.0508x; 1.0454x over previous
import functools

import jax
import jax.numpy as jnp
import numpy as np
from jax import lax
from jax.experimental import pallas as pl
from jax.experimental.pallas import tpu as pltpu
from jax.experimental.pallas import tpu_sc as plsc

VOCAB = 100000
D = 768
B = 4
S = 2048
N_ROWS = B * S

_info = plsc.get_sparse_core_info()
NC, NS, L = _info.num_cores, _info.num_subcores, _info.num_lanes
NW = NC * NS
POS_PER_W = S // NW
BLK = 16
NBLK = POS_PER_W // BLK
GROUPS = D // L

SCALE = np.float32(np.sqrt(np.float32(D)))

_mesh = plsc.VectorSubcoreMesh(core_axis_name="c", subcore_axis_name="s")


@functools.partial(
    pl.kernel,
    mesh=_mesh,
    out_type=jax.ShapeDtypeStruct((N_ROWS, D), jnp.float32),
    scratch_types=(
        [pltpu.VMEM((B * POS_PER_W,), jnp.int32)]
        + [pltpu.VMEM((BLK, D), jnp.float32) for _ in range(2 * (B + 1))]
        + [pltpu.SemaphoreType.DMA for _ in range(5)]
    ),
)
def _embed_kernel(
    idx_hbm, table_hbm, pos_hbm, out_hbm,
    idx_v,
    xa0, xa1, xa2, xa3, pa,
    xb0, xb1, xb2, xb3, pb,
    isem, gin0, gin1, gout0, gout1,
):
    wid = lax.axis_index("s") * NC + lax.axis_index("c")
    pos0 = wid * POS_PER_W

    icps = [
        pltpu.async_copy(
            idx_hbm.at[b, pl.ds(pos0, POS_PER_W)],
            idx_v.at[pl.ds(b * POS_PER_W, POS_PER_W)],
            isem,
        )
        for b in range(B)
    ]
    for icp in icps:
        icp.wait()

    xv = ((xa0, xa1, xa2, xa3), (xb0, xb1, xb2, xb3))
    pv = (pa, pb)
    gin = (gin0, gin1)
    gout = (gout0, gout1)

    def start_ins(g):
        p = g % 2
        cps = [
            pltpu.async_copy(
                table_hbm.at[idx_v.at[pl.ds(b * POS_PER_W + g * BLK, BLK)]],
                xv[p][b],
                gin[p],
            )
            for b in range(B)
        ]
        cps.append(
            pltpu.async_copy(
                pos_hbm.at[pl.ds(pos0 + g * BLK, BLK), :], pv[p], gin[p]
            )
        )
        return cps

    def start_outs(g):
        p = g % 2
        return [
            pltpu.async_copy(
                xv[p][b],
                out_hbm.at[pl.ds(b * S + pos0 + g * BLK, BLK), :],
                gout[p],
            )
            for b in range(B)
        ]

    pending_in = {0: start_ins(0)}
    pending_out = {}

    for g in range(NBLK):
        p = g % 2
        for cp in pending_in.pop(g):
            cp.wait()
        if g + 1 < NBLK:
            q = (g + 1) % 2
            if q in pending_out:
                for cp in pending_out.pop(q):
                    cp.wait()
            pending_in[g + 1] = start_ins(g + 1)

        x0, x1, x2, x3 = xv[p]
        pos_v = pv[p]

        @plsc.parallel_loop(0, BLK, unroll=1)
        def row_body(r):
            @plsc.parallel_loop(0, D, step=L, unroll=4)
            def group_body(off):
                sl = pl.ds(off, L)
                pg = pos_v[r, sl]
                x0[r, sl] = x0[r, sl] * SCALE + pg
                x1[r, sl] = x1[r, sl] * SCALE + pg
                x2[r, sl] = x2[r, sl] * SCALE + pg
                x3[r, sl] = x3[r, sl] * SCALE + pg

        pending_out[p] = start_outs(g)

    for p in list(pending_out):
        for cp in pending_out.pop(p):
            cp.wait()


def kernel(inputs, token_table, pos_embedding):
    idx = inputs.astype(jnp.int32)
    out = _embed_kernel(idx, token_table, pos_embedding)
    return out.reshape(B, S, D)


# speedup vs baseline: 1.0529x; 1.0020x over previous
import functools

import jax
import jax.numpy as jnp
import numpy as np
from jax import lax
from jax.experimental import pallas as pl
from jax.experimental.pallas import tpu as pltpu
from jax.experimental.pallas import tpu_sc as plsc

VOCAB = 100000
D = 768
B = 4
S = 2048
N_ROWS = B * S

_info = plsc.get_sparse_core_info()
NC, NS, L = _info.num_cores, _info.num_subcores, _info.num_lanes
NW = NC * NS
POS_PER_W = S // NW
BLK = 8
NBLK = POS_PER_W // BLK
NGEN = 4
AHEAD = 2
ROWS = B * BLK
GROUPS = D // L

SCALE = np.float32(np.sqrt(np.float32(D)))

_mesh = plsc.VectorSubcoreMesh(core_axis_name="c", subcore_axis_name="s")


@functools.partial(
    pl.kernel,
    mesh=_mesh,
    out_type=jax.ShapeDtypeStruct((N_ROWS, D), jnp.float32),
    scratch_types=(
        [pltpu.VMEM((NBLK * ROWS,), jnp.int32)]
        + [pltpu.VMEM((ROWS, D), jnp.float32) for _ in range(NGEN)]
        + [pltpu.VMEM((BLK, D), jnp.float32) for _ in range(NGEN)]
        + [pltpu.SemaphoreType.DMA for _ in range(1 + 2 * NGEN)]
    ),
)
def _embed_kernel(
    idx_hbm, table_hbm, pos_hbm, out_hbm,
    idx_v, xa, xb, xc, xd, pa, pb, pc, pd,
    isem, gin0, gin1, gin2, gin3, gout0, gout1, gout2, gout3,
):
    wid = lax.axis_index("s") * NC + lax.axis_index("c")
    pos0 = wid * POS_PER_W

    icp = pltpu.async_copy(
        idx_hbm.at[pl.ds(wid * (NBLK * ROWS), NBLK * ROWS)], idx_v, isem
    )

    xv = (xa, xb, xc, xd)
    pv = (pa, pb, pc, pd)
    gin = (gin0, gin1, gin2, gin3)
    gout = (gout0, gout1, gout2, gout3)

    def start_ins(g):
        p = g % NGEN
        return [
            pltpu.async_copy(
                pos_hbm.at[pl.ds(pos0 + g * BLK, BLK), :], pv[p], gin[p]
            ),
            pltpu.async_copy(
                table_hbm.at[idx_v.at[pl.ds(g * ROWS, ROWS)]], xv[p], gin[p]
            ),
        ]

    def start_outs(g):
        p = g % NGEN
        return [
            pltpu.async_copy(
                xv[p].at[pl.ds(b * BLK, BLK), :],
                out_hbm.at[pl.ds(b * S + pos0 + g * BLK, BLK), :],
                gout[p],
            )
            for b in range(B)
        ]

    icp.wait()

    pending_in = {g: start_ins(g) for g in range(AHEAD)}
    pending_out = {}

    for g in range(NBLK):
        p = g % NGEN
        for cp in pending_in.pop(g):
            cp.wait()
        if g + AHEAD < NBLK:
            q = (g + AHEAD) % NGEN
            if q in pending_out:
                for cp in pending_out.pop(q):
                    cp.wait()
            pending_in[g + AHEAD] = start_ins(g + AHEAD)

        x_v = xv[p]
        pos_v = pv[p]

        @plsc.parallel_loop(0, BLK, unroll=1)
        def row_body(r):
            @plsc.parallel_loop(0, D, step=L, unroll=4)
            def group_body(off):
                sl = pl.ds(off, L)
                pg = pos_v[r, sl]
                x_v[r, sl] = x_v[r, sl] * SCALE + pg
                x_v[BLK + r, sl] = x_v[BLK + r, sl] * SCALE + pg
                x_v[2 * BLK + r, sl] = x_v[2 * BLK + r, sl] * SCALE + pg
                x_v[3 * BLK + r, sl] = x_v[3 * BLK + r, sl] * SCALE + pg

        pending_out[p] = start_outs(g)

    for p in list(pending_out):
        for cp in pending_out.pop(p):
            cp.wait()


def kernel(inputs, token_table, pos_embedding):
    idx = (
        inputs.astype(jnp.int32)
        .reshape(B, NW, NBLK, BLK)
        .transpose(1, 2, 0, 3)
        .reshape(N_ROWS)
    )
    out = _embed_kernel(idx, token_table, pos_embedding)
    return out.reshape(B, S, D)
